# baseline (speedup 1.0000x reference)
import functools

import jax
import jax.numpy as jnp
from jax import lax
from jax.experimental import pallas as pl
from jax.experimental.pallas import tpu as pltpu
from jax.experimental.pallas import tpu_sc as plsc


def _mm_body(enc_ref, w_ref, b_ref, m_ref):
    m_ref[...] = lax.dot_general(
        enc_ref[...], w_ref[...],
        dimension_numbers=(((1,), (1,)), ((), ())),
        preferred_element_type=jnp.float32,
    ) + b_ref[...]


def _fused_table(enc_table, dec_w, dec_b2d):
    v_enc, _ = enc_table.shape
    v_dec, _ = dec_w.shape
    return pl.pallas_call(
        _mm_body,
        out_shape=jax.ShapeDtypeStruct((v_enc, v_dec), jnp.float32),
    )(enc_table, dec_w, dec_b2d)


@functools.lru_cache(maxsize=None)
def _make_gather(batch: int, seq: int, vocab: int):
    info = plsc.get_sparse_core_info()
    nw = info.num_cores * info.num_subcores
    assert batch % nw == 0
    b_per_w = batch // nw
    assert b_per_w % 2 == 0
    n_pairs = b_per_w // 2
    mesh = plsc.VectorSubcoreMesh(core_axis_name="c", subcore_axis_name="s")

    @functools.partial(
        pl.kernel,
        mesh=mesh,
        out_type=jax.ShapeDtypeStruct((batch, seq, vocab), jnp.float32),
        scratch_types=[
            pltpu.VMEM((b_per_w, seq), jnp.int32),
            pltpu.VMEM((seq, vocab), jnp.float32),
            pltpu.VMEM((seq, vocab), jnp.float32),
            pltpu.SemaphoreType.DMA,
            pltpu.SemaphoreType.DMA,
            pltpu.SemaphoreType.DMA,
            pltpu.SemaphoreType.DMA,
        ],
        compiler_params=pltpu.CompilerParams(use_tc_tiling_on_sc=False),
    )
    def gather_k(m_hbm, idx_hbm, out_hbm, idx_v, buf0, buf1,
                 gs0, gs1, ws0, ws1):
        wid = lax.axis_index("s") * info.num_cores + lax.axis_index("c")
        base = wid * b_per_w
        bufs, gsems, wsems = (buf0, buf1), (gs0, gs1), (ws0, ws1)
        pltpu.sync_copy(idx_hbm.at[pl.ds(base, b_per_w)], idx_v)

        def start_gather(g, p):
            pltpu.make_async_copy(
                m_hbm.at[idx_v.at[g]], bufs[p], gsems[p]).start()

        def wait_gather(p):
            pltpu.make_async_copy(
                m_hbm.at[pl.ds(0, seq)], bufs[p], gsems[p]).wait()

        def start_wb(g, p):
            pltpu.make_async_copy(
                bufs[p], out_hbm.at[base + g], wsems[p]).start()

        def wait_wb(p):
            pltpu.make_async_copy(
                bufs[p], out_hbm.at[base], wsems[p]).wait()

        start_gather(0, 0)
        start_gather(1, 1)

        def body(t, carry):
            g0 = 2 * t
            for p in (0, 1):
                wait_gather(p)
                start_wb(g0 + p, p)
            for p in (0, 1):
                @pl.when(t < n_pairs - 1)
                def _():
                    wait_wb(p)
                    start_gather(g0 + 2 + p, p)
            return carry

        lax.fori_loop(0, n_pairs, body, 0)
        wait_wb(0)
        wait_wb(1)

    return gather_k


def kernel(_input, enc_table, dec_w, dec_b):
    b, l = _input.shape
    vocab = dec_w.shape[0]
    m = _fused_table(enc_table, dec_w, dec_b.reshape(1, -1))
    return _make_gather(b, l, vocab)(m, _input)


# speedup vs baseline: 1.0967x; 1.0967x over previous
import functools

import jax
import jax.numpy as jnp
from jax import lax
from jax.experimental import pallas as pl
from jax.experimental.pallas import tpu as pltpu
from jax.experimental.pallas import tpu_sc as plsc


@functools.lru_cache(maxsize=None)
def _make_sc_gather(n_tokens: int, embed: int, chunk: int):
    info = plsc.get_sparse_core_info()
    nw = info.num_cores * info.num_subcores
    assert n_tokens % nw == 0
    per_w = n_tokens // nw
    assert per_w % (2 * chunk) == 0 and chunk <= 128 and chunk % 8 == 0
    n_pairs = per_w // (2 * chunk)
    mesh = plsc.VectorSubcoreMesh(core_axis_name="c", subcore_axis_name="s")

    @functools.partial(
        pl.kernel,
        mesh=mesh,
        out_type=jax.ShapeDtypeStruct((n_tokens, embed), jnp.float32),
        scratch_types=[
            pltpu.VMEM((per_w,), jnp.int32),
            pltpu.VMEM((chunk, embed), jnp.float32),
            pltpu.VMEM((chunk, embed), jnp.float32),
            pltpu.SemaphoreType.DMA,
            pltpu.SemaphoreType.DMA,
            pltpu.SemaphoreType.DMA,
            pltpu.SemaphoreType.DMA,
        ],
    )
    def gather_k(tab_hbm, idx_hbm, out_hbm, idx_v, buf0, buf1,
                 gs0, gs1, ws0, ws1):
        wid = lax.axis_index("s") * info.num_cores + lax.axis_index("c")
        base = wid * per_w
        bufs, gsems, wsems = (buf0, buf1), (gs0, gs1), (ws0, ws1)
        pltpu.sync_copy(idx_hbm.at[pl.ds(base, per_w)], idx_v)

        def start_gather(g, p):
            pltpu.make_async_copy(
                tab_hbm.at[idx_v.at[pl.ds(g * chunk, chunk)]],
                bufs[p], gsems[p]).start()

        def wait_gather(p):
            pltpu.make_async_copy(
                tab_hbm.at[pl.ds(0, chunk)], bufs[p], gsems[p]).wait()

        def start_wb(g, p):
            pltpu.make_async_copy(
                bufs[p], out_hbm.at[pl.ds(base + g * chunk, chunk)],
                wsems[p]).start()

        def wait_wb(p):
            pltpu.make_async_copy(
                bufs[p], out_hbm.at[pl.ds(base, chunk)], wsems[p]).wait()

        start_gather(0, 0)
        start_gather(1, 1)

        def body(t, carry):
            g0 = 2 * t
            for p in (0, 1):
                wait_gather(p)
                start_wb(g0 + p, p)
            for p in (0, 1):
                @pl.when(t < n_pairs - 1)
                def _():
                    wait_wb(p)
                    start_gather(g0 + 2 + p, p)
            return carry

        lax.fori_loop(0, n_pairs, body, 0)
        wait_wb(0)
        wait_wb(1)

    return gather_k


def _make_decoder(batch: int, seq: int, seq_pad: int, embed: int, vocab: int,
                  bb: int):
    assert batch % bb == 0
    grid = (batch // bb,)
    rows = bb * seq_pad

    def body(emb_ref, w_ref, b_ref, out_ref):
        y = lax.dot_general(
            emb_ref[...], w_ref[...],
            dimension_numbers=(((1,), (1,)), ((), ())),
            preferred_element_type=jnp.float32,
        ) + b_ref[...]
        for i in range(bb):
            out_ref[i] = y[i * seq_pad:i * seq_pad + seq]

    return pl.pallas_call(
        body,
        grid=grid,
        in_specs=[
            pl.BlockSpec((rows, embed), lambda i: (i, 0)),
            pl.BlockSpec((vocab, embed), lambda i: (0, 0)),
            pl.BlockSpec((1, vocab), lambda i: (0, 0)),
        ],
        out_specs=pl.BlockSpec((bb, seq, vocab), lambda i: (i, 0, 0)),
        out_shape=jax.ShapeDtypeStruct((batch, seq, vocab), jnp.float32),
    )


def kernel(_input, enc_table, dec_w, dec_b):
    b, l = _input.shape
    vocab, embed = dec_w.shape
    l_pad = (l + 7) // 8 * 8
    idx = jnp.pad(_input, ((0, 0), (0, l_pad - l))).reshape(-1)
    emb = _make_sc_gather(b * l_pad, embed, 112)(enc_table, idx)
    return _make_decoder(b, l, l_pad, embed, vocab, 16)(
        emb, dec_w, dec_b.reshape(1, -1))


# speedup vs baseline: 1.1165x; 1.0180x over previous
import functools

import jax
import jax.numpy as jnp
from jax import lax
from jax.experimental import pallas as pl
from jax.experimental.pallas import tpu as pltpu
from jax.experimental.pallas import tpu_sc as plsc


@functools.lru_cache(maxsize=None)
def _make_sc_gather(n_tokens: int, embed: int, chunk: int):
    info = plsc.get_sparse_core_info()
    nw = info.num_cores * info.num_subcores
    assert n_tokens % nw == 0
    per_w = n_tokens // nw
    assert per_w % (2 * chunk) == 0 and chunk <= 128 and chunk % 8 == 0
    n_pairs = per_w // (2 * chunk)
    mesh = plsc.VectorSubcoreMesh(core_axis_name="c", subcore_axis_name="s")

    @functools.partial(
        pl.kernel,
        mesh=mesh,
        out_type=jax.ShapeDtypeStruct((n_tokens, embed), jnp.float32),
        scratch_types=[
            pltpu.VMEM((per_w,), jnp.int32),
            pltpu.VMEM((chunk, embed), jnp.float32),
            pltpu.VMEM((chunk, embed), jnp.float32),
            pltpu.SemaphoreType.DMA,
            pltpu.SemaphoreType.DMA,
            pltpu.SemaphoreType.DMA,
            pltpu.SemaphoreType.DMA,
        ],
    )
    def gather_k(tab_hbm, idx_hbm, out_hbm, idx_v, buf0, buf1,
                 gs0, gs1, ws0, ws1):
        wid = lax.axis_index("s") * info.num_cores + lax.axis_index("c")
        base = wid * per_w
        bufs, gsems, wsems = (buf0, buf1), (gs0, gs1), (ws0, ws1)
        pltpu.sync_copy(idx_hbm.at[pl.ds(base, per_w)], idx_v)

        def start_gather(g, p):
            pltpu.make_async_copy(
                tab_hbm.at[idx_v.at[pl.ds(g * chunk, chunk)]],
                bufs[p], gsems[p]).start()

        def wait_gather(p):
            pltpu.make_async_copy(
                tab_hbm.at[pl.ds(0, chunk)], bufs[p], gsems[p]).wait()

        def start_wb(g, p):
            pltpu.make_async_copy(
                bufs[p], out_hbm.at[pl.ds(base + g * chunk, chunk)],
                wsems[p]).start()

        def wait_wb(p):
            pltpu.make_async_copy(
                bufs[p], out_hbm.at[pl.ds(base, chunk)], wsems[p]).wait()

        start_gather(0, 0)
        start_gather(1, 1)

        def body(t, carry):
            g0 = 2 * t
            for p in (0, 1):
                wait_gather(p)
                start_wb(g0 + p, p)
            for p in (0, 1):
                @pl.when(t < n_pairs - 1)
                def _():
                    wait_wb(p)
                    start_gather(g0 + 2 + p, p)
            return carry

        lax.fori_loop(0, n_pairs, body, 0)
        wait_wb(0)
        wait_wb(1)

    return gather_k


def _make_decoder(batch: int, seq: int, seq_pad: int, embed: int, vocab: int,
                  bb: int):
    assert batch % bb == 0
    grid = (batch // bb,)
    rows = bb * seq_pad

    def body(emb_ref, w_ref, b_ref, out_ref):
        y = lax.dot_general(
            emb_ref[...], w_ref[...],
            dimension_numbers=(((1,), (1,)), ((), ())),
            preferred_element_type=jnp.float32,
        ) + b_ref[...]
        out_ref[...] = y

    return pl.pallas_call(
        body,
        grid=grid,
        in_specs=[
            pl.BlockSpec((rows, embed), lambda i: (i, 0)),
            pl.BlockSpec((vocab, embed), lambda i: (0, 0)),
            pl.BlockSpec((1, vocab), lambda i: (0, 0)),
        ],
        out_specs=pl.BlockSpec((rows, vocab), lambda i: (i, 0)),
        out_shape=jax.ShapeDtypeStruct((batch // bb * rows, vocab),
                                       jnp.float32),
    )


def kernel(_input, enc_table, dec_w, dec_b):
    b, l = _input.shape
    vocab, embed = dec_w.shape
    l_pad = (l + 7) // 8 * 8
    idx = jnp.pad(_input, ((0, 0), (0, l_pad - l))).reshape(-1)
    emb = _make_sc_gather(b * l_pad, embed, 112)(enc_table, idx)
    return _make_decoder(b, l, l_pad, embed, vocab, 16)(
        emb, dec_w, dec_b.reshape(1, -1))


# speedup vs baseline: 5.4503x; 4.8816x over previous
import functools

import jax
import jax.numpy as jnp
from jax import lax
from jax.experimental import pallas as pl
from jax.experimental.pallas import tpu as pltpu
from jax.experimental.pallas import tpu_sc as plsc

_NBUF = 4


@functools.lru_cache(maxsize=None)
def _make_sc_gather(n_tokens: int, embed: int, chunk: int):
    info = plsc.get_sparse_core_info()
    nw = info.num_cores * info.num_subcores
    assert n_tokens % nw == 0
    per_w = n_tokens // nw
    assert per_w % (_NBUF * chunk) == 0 and chunk <= 128 and chunk % 8 == 0
    n_rounds = per_w // (_NBUF * chunk)
    mesh = plsc.VectorSubcoreMesh(core_axis_name="c", subcore_axis_name="s")

    @functools.partial(
        pl.kernel,
        mesh=mesh,
        out_type=jax.ShapeDtypeStruct((n_tokens, embed), jnp.float32),
        scratch_types=(
            [pltpu.VMEM((per_w,), jnp.int32)]
            + [pltpu.VMEM((chunk, embed), jnp.float32)] * _NBUF
            + [pltpu.SemaphoreType.DMA] * (2 * _NBUF)
        ),
    )
    def gather_k(tab_hbm, idx_hbm, out_hbm, idx_v, *bufs_sems):
        bufs = bufs_sems[:_NBUF]
        gsems = bufs_sems[_NBUF:2 * _NBUF]
        wsems = bufs_sems[2 * _NBUF:]
        wid = lax.axis_index("s") * info.num_cores + lax.axis_index("c")
        base = wid * per_w
        pltpu.sync_copy(idx_hbm.at[pl.ds(base, per_w)], idx_v)

        def start_gather(g, p):
            pltpu.make_async_copy(
                tab_hbm.at[idx_v.at[pl.ds(g * chunk, chunk)]],
                bufs[p], gsems[p]).start()

        def wait_gather(p):
            pltpu.make_async_copy(
                tab_hbm.at[pl.ds(0, chunk)], bufs[p], gsems[p]).wait()

        def start_wb(g, p):
            pltpu.make_async_copy(
                bufs[p], out_hbm.at[pl.ds(base + g * chunk, chunk)],
                wsems[p]).start()

        def wait_wb(p):
            pltpu.make_async_copy(
                bufs[p], out_hbm.at[pl.ds(base, chunk)], wsems[p]).wait()

        for p in range(_NBUF):
            start_gather(p, p)

        def body(t, carry):
            g0 = _NBUF * t
            for p in range(_NBUF):
                wait_gather(p)
                start_wb(g0 + p, p)
            for p in range(_NBUF):
                @pl.when(t < n_rounds - 1)
                def _():
                    wait_wb(p)
                    start_gather(g0 + _NBUF + p, p)
            return carry

        lax.fori_loop(0, n_rounds, body, 0)
        for p in range(_NBUF):
            wait_wb(p)

    return gather_k


def _make_decoder(batch: int, seq: int, embed: int, vocab: int, lb: int):
    assert seq % lb == 0
    grid = (seq // lb,)
    rows = lb * batch

    def body(emb_ref, w_ref, b_ref, out_ref):
        y = lax.dot_general(
            w_ref[...], emb_ref[...],
            dimension_numbers=(((1,), (1,)), ((), ())),
            preferred_element_type=jnp.float32,
        ) + b_ref[...]
        for j in range(lb):
            out_ref[j] = y[:, j * batch:(j + 1) * batch]

    return pl.pallas_call(
        body,
        grid=grid,
        in_specs=[
            pl.BlockSpec((rows, embed), lambda i: (i, 0)),
            pl.BlockSpec((vocab, embed), lambda i: (0, 0)),
            pl.BlockSpec((vocab, 1), lambda i: (0, 0)),
        ],
        out_specs=pl.BlockSpec((lb, vocab, batch), lambda i: (i, 0, 0)),
        out_shape=jax.ShapeDtypeStruct((seq, vocab, batch), jnp.float32),
    )


def kernel(_input, enc_table, dec_w, dec_b):
    b, l = _input.shape
    vocab, embed = dec_w.shape
    idx_lm = _input.T.reshape(-1)
    emb = _make_sc_gather(b * l, embed, 80)(enc_table, idx_lm)
    out_t = _make_decoder(b, l, embed, vocab, 2)(
        emb, dec_w, dec_b.reshape(-1, 1))
    return jnp.transpose(out_t, (2, 0, 1))
